# Initial kernel scaffold; baseline (speedup 1.0000x reference)
import jax
import jax.numpy as jnp
from jax.experimental import pallas as pl


def kernel(x_user, x_job, edge_uj, edge_ju, params):
    raise NotImplementedError("write your pallas kernel here")


# speedup vs baseline: 13.1911x; 13.1911x over previous
import functools

import jax
import jax.numpy as jnp
import numpy as np
from jax import lax
from jax.experimental import pallas as pl
from jax.experimental.pallas import tpu as pltpu
from jax.experimental.pallas import tpu_sc as plsc

N = 10000
E = 320000
HID = 128
OUT = 64
H = 8
DH = 16
L = 2
NC, NS = 2, 16
EPT = E // NS
CH = 80
NCHUNK = EPT // CH
NPAD = 10240
NPT = NPAD // NS
RB = 1000
GRID = 2 * N // RB

_f32 = jnp.float32


def _mm_body(x_ref, w_ref, b_ref, o_ref, *, act):
    y = jnp.dot(x_ref[...], w_ref[0], preferred_element_type=_f32) + b_ref[0, 0]
    if act == "relu":
        y = jnp.maximum(y, 0.0)
    o_ref[...] = y


def _typed_matmul(x, w2, b2, nout, act):
    return pl.pallas_call(
        functools.partial(_mm_body, act=act),
        grid=(GRID,),
        in_specs=[
            pl.BlockSpec((RB, HID), lambda i: (i, 0)),
            pl.BlockSpec((1, HID, nout), lambda i: (i // (GRID // 2), 0, 0)),
            pl.BlockSpec((1, 1, nout), lambda i: (i // (GRID // 2), 0, 0)),
        ],
        out_specs=pl.BlockSpec((RB, nout), lambda i: (i, 0)),
        out_shape=jax.ShapeDtypeStruct((2 * N, nout), _f32),
    )(x, w2, b2.reshape(2, 1, nout))


def _qkv_body(x_ref, w_ref, b_ref, q_ref, k_ref, v_ref):
    y = jnp.dot(x_ref[...], w_ref[0], preferred_element_type=_f32) + b_ref[0, 0]
    q_ref[...] = y[:, :HID]
    k_ref[...] = y[:, HID:2 * HID]
    v_ref[...] = y[:, 2 * HID:]


def _qkv(x, w2, b2):
    outs = [jax.ShapeDtypeStruct((2 * N, HID), _f32)] * 3
    return pl.pallas_call(
        _qkv_body,
        grid=(GRID,),
        in_specs=[
            pl.BlockSpec((RB, HID), lambda i: (i, 0)),
            pl.BlockSpec((1, HID, 3 * HID), lambda i: (i // (GRID // 2), 0, 0)),
            pl.BlockSpec((1, 1, 3 * HID), lambda i: (i // (GRID // 2), 0, 0)),
        ],
        out_specs=[pl.BlockSpec((RB, HID), lambda i: (i, 0))] * 3,
        out_shape=outs,
    )(x, w2, b2.reshape(2, 1, 3 * HID))


def _finish_body(m_ref, d_ref, x_ref, sel_ref, aw_ref, ab_ref, bt_ref, o_ref):
    m = m_ref[0]
    den = d_ref[0]
    denb = jnp.dot(den, sel_ref[...], preferred_element_type=_f32) + 1e-16
    msg = m / denb
    hmid = jax.nn.gelu(msg)
    y = jnp.dot(hmid, aw_ref[0], preferred_element_type=_f32) + ab_ref[0, 0]
    beta = bt_ref[0, 0, 0]
    o_ref[...] = jnp.maximum(beta * y + (1.0 - beta) * x_ref[...], 0.0)


def _finish(msum, dsum, x, sel, aw2, ab2, beta2):
    half = GRID // 2
    return pl.pallas_call(
        _finish_body,
        grid=(GRID,),
        in_specs=[
            pl.BlockSpec((1, RB, HID), lambda i: (1 - i // half, i % half, 0)),
            pl.BlockSpec((1, RB, 16), lambda i: (1 - i // half, i % half, 0)),
            pl.BlockSpec((RB, HID), lambda i: (i, 0)),
            pl.BlockSpec((16, HID), lambda i: (0, 0)),
            pl.BlockSpec((1, HID, HID), lambda i: (i // half, 0, 0)),
            pl.BlockSpec((1, 1, HID), lambda i: (i // half, 0, 0)),
            pl.BlockSpec((1, 1, 1), lambda i: (i // half, 0, 0)),
        ],
        out_specs=pl.BlockSpec((RB, HID), lambda i: (i, 0)),
        out_shape=jax.ShapeDtypeStruct((2 * N, HID), _f32),
    )(msum, dsum, x, sel, aw2, ab2.reshape(2, 1, HID), beta2.reshape(2, 1, 1))


def _out_body(x_ref, w_ref, b_ref, o_ref):
    o_ref[...] = jnp.dot(x_ref[...], w_ref[...], preferred_element_type=_f32) + b_ref[...]


def _out_proj(x, w, b):
    return pl.pallas_call(
        _out_body,
        grid=(GRID,),
        in_specs=[
            pl.BlockSpec((RB, HID), lambda i: (i, 0)),
            pl.BlockSpec((HID, OUT), lambda i: (0, 0)),
            pl.BlockSpec((1, OUT), lambda i: (0, 0)),
        ],
        out_specs=pl.BlockSpec((RB, OUT), lambda i: (i, 0)),
        out_shape=jax.ShapeDtypeStruct((2 * N, OUT), _f32),
    )(x, w, b.reshape(1, OUT))


def _edge_kernel(q_hbm, k_hbm, v_hbm, src_hbm, dst_hbm, outm_hbm, outd_hbm,
                 srcb, dstb, dstqb, qb, kb, vb, exb, accm_sh, accd_sh,
                 sem0, sem1, sem2):
    c = lax.axis_index("c")
    s = lax.axis_index("s")
    zrow = jnp.zeros((16,), _f32)

    def _vb_zero(r, _):
        for j in range(HID // 16):
            vb[r, pl.ds(j * 16, 16)] = zrow
        exb[r, pl.ds(0, 16)] = zrow
        return 0
    lax.fori_loop(0, CH, _vb_zero, 0)
    row0 = s * NPT
    for t in range(NPT // CH):
        pltpu.sync_copy(vb, accm_sh.at[pl.ds(row0 + t * CH, CH)])
        pltpu.sync_copy(exb, accd_sh.at[pl.ds(row0 + t * CH, CH)])

    plsc.subcore_barrier()

    src_off = c * N
    dst_off = (1 - c) * N
    ebase0 = c * E + s * EPT
    lanes = lax.iota(jnp.int32, 16)

    def chunk_body(ck, _):
        ebase = ebase0 + ck * CH
        pltpu.sync_copy(src_hbm.at[pl.ds(ebase, CH)], srcb)
        pltpu.sync_copy(dst_hbm.at[pl.ds(ebase, CH)], dstb)
        for i in range(CH // 16):
            sl = pl.ds(i * 16, 16)
            srcb[sl] = srcb[sl] + src_off
            dstqb[sl] = dstb[sl] + dst_off
        cp0 = pltpu.async_copy(q_hbm.at[dstqb], qb, sem0)
        cp1 = pltpu.async_copy(k_hbm.at[srcb], kb, sem1)
        cp2 = pltpu.async_copy(v_hbm.at[srcb], vb, sem2)
        cp0.wait(); cp1.wait(); cp2.wait()

        def group_body(g, _):
            rows = g * 16 + lanes
            for h in range(H):
                acc = jnp.zeros((16,), _f32)
                for d in range(DH):
                    col = jnp.full((16,), h * DH + d, jnp.int32)
                    qv = plsc.load_gather(qb, [rows, col])
                    kv = plsc.load_gather(kb, [rows, col])
                    acc = acc + qv * kv
                ex = jnp.exp(acc)
                plsc.store_scatter(exb, [rows, jnp.full((16,), h, jnp.int32)], ex)
                for d in range(DH):
                    col = jnp.full((16,), h * DH + d, jnp.int32)
                    vv = plsc.load_gather(vb, [rows, col])
                    plsc.store_scatter(vb, [rows, col], vv * ex)
            return 0
        lax.fori_loop(0, CH // 16, group_body, 0)

        pltpu.sync_copy(vb, accm_sh.at[dstb], add=True)
        pltpu.sync_copy(exb, accd_sh.at[dstb], add=True)
        return 0

    lax.fori_loop(0, NCHUNK, chunk_body, 0)

    plsc.subcore_barrier()
    pltpu.sync_copy(accm_sh.at[pl.ds(row0, NPT)],
                    outm_hbm.at[c, pl.ds(row0, NPT)])
    pltpu.sync_copy(accd_sh.at[pl.ds(row0, NPT)],
                    outd_hbm.at[c, pl.ds(row0, NPT)])


def _edge_agg(q_all, k_all, v_all, src_all, dst_all):
    mesh = plsc.VectorSubcoreMesh(core_axis_name="c", subcore_axis_name="s")
    kern = functools.partial(
        pl.kernel,
        mesh=mesh,
        compiler_params=pltpu.CompilerParams(
            needs_layout_passes=False, use_tc_tiling_on_sc=False),
        out_type=[jax.ShapeDtypeStruct((2, NPAD, HID), _f32),
                  jax.ShapeDtypeStruct((2, NPAD, 16), _f32)],
        scratch_types=[
            pltpu.VMEM((CH,), jnp.int32),
            pltpu.VMEM((CH,), jnp.int32),
            pltpu.VMEM((CH,), jnp.int32),
            pltpu.VMEM((CH, HID), _f32),
            pltpu.VMEM((CH, HID), _f32),
            pltpu.VMEM((CH, HID), _f32),
            pltpu.VMEM((CH, 16), _f32),
            pltpu.VMEM_SHARED((NPAD, HID), _f32),
            pltpu.VMEM_SHARED((NPAD, 16), _f32),
            pltpu.SemaphoreType.DMA,
            pltpu.SemaphoreType.DMA,
            pltpu.SemaphoreType.DMA,
        ],
    )(_edge_kernel)
    return kern(q_all, k_all, v_all, src_all, dst_all)


def _fuse_rel(w, b, rel, scale):
    bd = rel * scale[:, None, None]
    BD = (bd[:, :, None, :] * jnp.eye(H, dtype=_f32)[:, None, :, None]
          ).reshape(HID, HID)
    return w @ BD, b @ BD


def kernel(x_user, x_job, edge_uj, edge_ju, params):
    p = params
    sel = np.zeros((16, HID), np.float32)
    for h in range(H):
        sel[h, h * DH:(h + 1) * DH] = 1.0
    sel = jnp.asarray(sel)

    x_cat = jnp.concatenate([x_user, x_job], axis=0)
    in_w = jnp.stack([p["in_user_w"], p["in_job_w"]])
    in_b = jnp.stack([p["in_user_b"], p["in_job_b"]])
    x = _typed_matmul(x_cat, in_w, in_b, HID, "relu")

    src_all = jnp.concatenate([edge_uj[0], edge_ju[0]])
    dst_all = jnp.concatenate([edge_uj[1], edge_ju[1]])

    for l in range(L):
        s_uj = p[f"l{l}_prel_uj"] / np.sqrt(DH).astype(np.float32)
        s_ju = p[f"l{l}_prel_ju"] / np.sqrt(DH).astype(np.float32)
        ones = jnp.ones((H,), _f32)
        kw_u, kb_u = _fuse_rel(p[f"l{l}_K_user_w"], p[f"l{l}_K_user_b"],
                               p[f"l{l}_arel_uj"], s_uj)
        vw_u, vb_u = _fuse_rel(p[f"l{l}_V_user_w"], p[f"l{l}_V_user_b"],
                               p[f"l{l}_mrel_uj"], ones)
        kw_j, kb_j = _fuse_rel(p[f"l{l}_K_job_w"], p[f"l{l}_K_job_b"],
                               p[f"l{l}_arel_ju"], s_ju)
        vw_j, vb_j = _fuse_rel(p[f"l{l}_V_job_w"], p[f"l{l}_V_job_b"],
                               p[f"l{l}_mrel_ju"], ones)
        w2 = jnp.stack([
            jnp.concatenate([p[f"l{l}_Q_user_w"], kw_u, vw_u], axis=1),
            jnp.concatenate([p[f"l{l}_Q_job_w"], kw_j, vw_j], axis=1),
        ])
        b2 = jnp.stack([
            jnp.concatenate([p[f"l{l}_Q_user_b"], kb_u, vb_u]),
            jnp.concatenate([p[f"l{l}_Q_job_b"], kb_j, vb_j]),
        ])
        q_all, k_all, v_all = _qkv(x, w2, b2)
        msum, dsum = _edge_agg(q_all, k_all, v_all, src_all, dst_all)
        aw2 = jnp.stack([p[f"l{l}_A_user_w"], p[f"l{l}_A_job_w"]])
        ab2 = jnp.stack([p[f"l{l}_A_user_b"], p[f"l{l}_A_job_b"]])
        beta2 = jax.nn.sigmoid(jnp.stack([p[f"l{l}_skip_user"],
                                          p[f"l{l}_skip_job"]])).reshape(2, 1)
        x = _finish(msum, dsum, x, sel, aw2, ab2, beta2)

    y = _out_proj(x, p["out_w"], p["out_b"])
    return (y[:N], y[N:])


# speedup vs baseline: 13.7673x; 1.0437x over previous
import functools

import jax
import jax.numpy as jnp
import numpy as np
from jax import lax
from jax.experimental import pallas as pl
from jax.experimental.pallas import tpu as pltpu
from jax.experimental.pallas import tpu_sc as plsc

N = 10000
E = 320000
HID = 128
OUT = 64
H = 8
DH = 16
L = 2
NC, NS = 2, 16
EPT = E // NS
CH = 48
NCHUNK = (EPT // CH) & ~1
CHT = EPT - NCHUNK * CH
NPAD = 10240
NPT = NPAD // NS
RB = 1000
GRID = 2 * N // RB

_f32 = jnp.float32


def _mm_body(x_ref, w_ref, b_ref, o_ref, *, act):
    y = jnp.dot(x_ref[...], w_ref[0], preferred_element_type=_f32) + b_ref[0, 0]
    if act == "relu":
        y = jnp.maximum(y, 0.0)
    o_ref[...] = y


def _typed_matmul(x, w2, b2, nout, act):
    return pl.pallas_call(
        functools.partial(_mm_body, act=act),
        grid=(GRID,),
        in_specs=[
            pl.BlockSpec((RB, HID), lambda i: (i, 0)),
            pl.BlockSpec((1, HID, nout), lambda i: (i // (GRID // 2), 0, 0)),
            pl.BlockSpec((1, 1, nout), lambda i: (i // (GRID // 2), 0, 0)),
        ],
        out_specs=pl.BlockSpec((RB, nout), lambda i: (i, 0)),
        out_shape=jax.ShapeDtypeStruct((2 * N, nout), _f32),
    )(x, w2, b2.reshape(2, 1, nout))


def _qkv_body(x_ref, w_ref, b_ref, q_ref, k_ref, v_ref):
    y = jnp.dot(x_ref[...], w_ref[0], preferred_element_type=_f32) + b_ref[0, 0]
    q_ref[...] = y[:, :HID]
    k_ref[...] = y[:, HID:2 * HID]
    v_ref[...] = y[:, 2 * HID:]


def _qkv(x, w2, b2):
    outs = [jax.ShapeDtypeStruct((2 * N, HID), _f32)] * 3
    return pl.pallas_call(
        _qkv_body,
        grid=(GRID,),
        in_specs=[
            pl.BlockSpec((RB, HID), lambda i: (i, 0)),
            pl.BlockSpec((1, HID, 3 * HID), lambda i: (i // (GRID // 2), 0, 0)),
            pl.BlockSpec((1, 1, 3 * HID), lambda i: (i // (GRID // 2), 0, 0)),
        ],
        out_specs=[pl.BlockSpec((RB, HID), lambda i: (i, 0))] * 3,
        out_shape=outs,
    )(x, w2, b2.reshape(2, 1, 3 * HID))


def _finish_body(m_ref, d_ref, x_ref, sel_ref, aw_ref, ab_ref, bt_ref, o_ref):
    m = m_ref[0]
    den = d_ref[0]
    denb = jnp.dot(den, sel_ref[...], preferred_element_type=_f32) + 1e-16
    msg = m / denb
    hmid = jax.nn.gelu(msg)
    y = jnp.dot(hmid, aw_ref[0], preferred_element_type=_f32) + ab_ref[0, 0]
    beta = bt_ref[0, 0, 0]
    o_ref[...] = jnp.maximum(beta * y + (1.0 - beta) * x_ref[...], 0.0)


def _finish(msum, dsum, x, sel, aw2, ab2, beta2):
    half = GRID // 2
    return pl.pallas_call(
        _finish_body,
        grid=(GRID,),
        in_specs=[
            pl.BlockSpec((1, RB, HID), lambda i: (1 - i // half, i % half, 0)),
            pl.BlockSpec((1, RB, 16), lambda i: (1 - i // half, i % half, 0)),
            pl.BlockSpec((RB, HID), lambda i: (i, 0)),
            pl.BlockSpec((16, HID), lambda i: (0, 0)),
            pl.BlockSpec((1, HID, HID), lambda i: (i // half, 0, 0)),
            pl.BlockSpec((1, 1, HID), lambda i: (i // half, 0, 0)),
            pl.BlockSpec((1, 1, 1), lambda i: (i // half, 0, 0)),
        ],
        out_specs=pl.BlockSpec((RB, HID), lambda i: (i, 0)),
        out_shape=jax.ShapeDtypeStruct((2 * N, HID), _f32),
    )(msum, dsum, x, sel, aw2, ab2.reshape(2, 1, HID), beta2.reshape(2, 1, 1))


def _out_body(x_ref, w_ref, b_ref, o_ref):
    o_ref[...] = jnp.dot(x_ref[...], w_ref[...], preferred_element_type=_f32) + b_ref[...]


def _out_proj(x, w, b):
    return pl.pallas_call(
        _out_body,
        grid=(GRID,),
        in_specs=[
            pl.BlockSpec((RB, HID), lambda i: (i, 0)),
            pl.BlockSpec((HID, OUT), lambda i: (0, 0)),
            pl.BlockSpec((1, OUT), lambda i: (0, 0)),
        ],
        out_specs=pl.BlockSpec((RB, OUT), lambda i: (i, 0)),
        out_shape=jax.ShapeDtypeStruct((2 * N, OUT), _f32),
    )(x, w, b.reshape(1, OUT))


def _edge_kernel(q_hbm, k_hbm, v_hbm, src_hbm, dst_hbm, outm_hbm, outd_hbm,
                 srcb0, dstb0, dstqb0, qb0, kb0, vb0, exb0,
                 srcb1, dstb1, dstqb1, qb1, kb1, vb1, exb1,
                 srcbt, dstbt, dstqbt,
                 accm_sh, accd_sh,
                 gsem0, gsem1, gsem2, gsem3, gsem4, gsem5):
    c = lax.axis_index("c")
    s = lax.axis_index("s")
    zrow = jnp.zeros((16,), _f32)
    sets = ((srcb0, dstb0, dstqb0, qb0, kb0, vb0, exb0, (gsem0, gsem1, gsem2)),
            (srcb1, dstb1, dstqb1, qb1, kb1, vb1, exb1, (gsem3, gsem4, gsem5)))

    def _vb_zero(r, _):
        for j in range(HID // 16):
            vb0[r, pl.ds(j * 16, 16)] = zrow
        exb0[r, pl.ds(0, 16)] = zrow
        return 0
    lax.fori_loop(0, CH, _vb_zero, 0)
    row0 = s * NPT
    nzc = NPT // CH
    for t in range(nzc):
        pltpu.sync_copy(vb0, accm_sh.at[pl.ds(row0 + t * CH, CH)])
        pltpu.sync_copy(exb0, accd_sh.at[pl.ds(row0 + t * CH, CH)])
    rem = NPT - nzc * CH
    if rem:
        pltpu.sync_copy(vb0.at[pl.ds(0, rem)],
                        accm_sh.at[pl.ds(row0 + nzc * CH, rem)])
        pltpu.sync_copy(exb0.at[pl.ds(0, rem)],
                        accd_sh.at[pl.ds(row0 + nzc * CH, rem)])

    plsc.subcore_barrier()

    src_off = c * N
    dst_off = (1 - c) * N
    ebase0 = c * E + s * EPT
    lanes = lax.iota(jnp.int32, 16)

    def issue(ck, st):
        srcb, dstb, dstqb, qb, kb, vb, exb, sems = st
        ebase = ebase0 + ck * CH
        pltpu.sync_copy(src_hbm.at[pl.ds(ebase, CH)], srcb)
        pltpu.sync_copy(dst_hbm.at[pl.ds(ebase, CH)], dstb)
        for i in range(CH // 16):
            sl = pl.ds(i * 16, 16)
            srcb[sl] = srcb[sl] + src_off
            dstqb[sl] = dstb[sl] + dst_off
        pltpu.async_copy(q_hbm.at[dstqb], qb, sems[0])
        pltpu.async_copy(k_hbm.at[srcb], kb, sems[1])
        pltpu.async_copy(v_hbm.at[srcb], vb, sems[2])

    def drain(st):
        srcb, dstb, dstqb, qb, kb, vb, exb, sems = st
        pltpu.make_async_copy(q_hbm.at[dstqb], qb, sems[0]).wait()
        pltpu.make_async_copy(k_hbm.at[srcb], kb, sems[1]).wait()
        pltpu.make_async_copy(v_hbm.at[srcb], vb, sems[2]).wait()

    def make_group_body(qb, kb, vb, exb):
        def group_body(g, _):
            rows = g * 16 + lanes
            for h in range(H):
                acc = jnp.zeros((16,), _f32)
                for d in range(DH):
                    col = jnp.full((16,), h * DH + d, jnp.int32)
                    qv = plsc.load_gather(qb, [rows, col])
                    kv = plsc.load_gather(kb, [rows, col])
                    acc = acc + qv * kv
                ex = jnp.exp(acc)
                plsc.store_scatter(exb, [rows, jnp.full((16,), h, jnp.int32)], ex)
                for d in range(DH):
                    col = jnp.full((16,), h * DH + d, jnp.int32)
                    vv = plsc.load_gather(vb, [rows, col])
                    plsc.store_scatter(vb, [rows, col], vv * ex)
            return 0
        return group_body

    issue(0, sets[0])

    def pair_body(j, _):
        for b in (0, 1):
            ck = 2 * j + b
            st = sets[b]
            drain(st)

            @pl.when(ck < NCHUNK - 1)
            def _():
                issue(ck + 1, sets[1 - b])
            lax.fori_loop(0, CH // 16, make_group_body(st[3], st[4], st[5], st[6]), 0)
            pltpu.sync_copy(st[5], accm_sh.at[st[1]], add=True)
            pltpu.sync_copy(st[6], accd_sh.at[st[1]], add=True)
        return 0
    lax.fori_loop(0, NCHUNK // 2, pair_body, 0)

    if CHT:
        ebase = ebase0 + NCHUNK * CH
        pltpu.sync_copy(src_hbm.at[pl.ds(ebase, CHT)], srcbt)
        pltpu.sync_copy(dst_hbm.at[pl.ds(ebase, CHT)], dstbt)
        for i in range(CHT // 16):
            sl = pl.ds(i * 16, 16)
            srcbt[sl] = srcbt[sl] + src_off
            dstqbt[sl] = dstbt[sl] + dst_off
        qbt, kbt, vbt = (qb0.at[pl.ds(0, CHT)], kb0.at[pl.ds(0, CHT)],
                         vb0.at[pl.ds(0, CHT)])
        exbt = exb0.at[pl.ds(0, CHT)]
        pltpu.async_copy(q_hbm.at[dstqbt], qbt, gsem0)
        pltpu.async_copy(k_hbm.at[srcbt], kbt, gsem1)
        pltpu.async_copy(v_hbm.at[srcbt], vbt, gsem2)
        pltpu.make_async_copy(q_hbm.at[dstqbt], qbt, gsem0).wait()
        pltpu.make_async_copy(k_hbm.at[srcbt], kbt, gsem1).wait()
        pltpu.make_async_copy(v_hbm.at[srcbt], vbt, gsem2).wait()
        lax.fori_loop(0, CHT // 16, make_group_body(qb0, kb0, vb0, exb0), 0)
        pltpu.sync_copy(vb0.at[pl.ds(0, CHT)], accm_sh.at[dstbt], add=True)
        pltpu.sync_copy(exb0.at[pl.ds(0, CHT)], accd_sh.at[dstbt], add=True)

    plsc.subcore_barrier()
    pltpu.sync_copy(accm_sh.at[pl.ds(row0, NPT)],
                    outm_hbm.at[c, pl.ds(row0, NPT)])
    pltpu.sync_copy(accd_sh.at[pl.ds(row0, NPT)],
                    outd_hbm.at[c, pl.ds(row0, NPT)])


def _edge_agg(q_all, k_all, v_all, src_all, dst_all):
    mesh = plsc.VectorSubcoreMesh(core_axis_name="c", subcore_axis_name="s")
    idx_t = pltpu.VMEM((CH,), jnp.int32)
    buf_set = [idx_t, idx_t, idx_t,
               pltpu.VMEM((CH, HID), _f32), pltpu.VMEM((CH, HID), _f32),
               pltpu.VMEM((CH, HID), _f32), pltpu.VMEM((CH, 16), _f32)]
    kern = functools.partial(
        pl.kernel,
        mesh=mesh,
        compiler_params=pltpu.CompilerParams(
            needs_layout_passes=False, use_tc_tiling_on_sc=False),
        out_type=[jax.ShapeDtypeStruct((2, NPAD, HID), _f32),
                  jax.ShapeDtypeStruct((2, NPAD, 16), _f32)],
        scratch_types=(buf_set + buf_set
                       + [pltpu.VMEM((CHT,), jnp.int32)] * 3
                       + [pltpu.VMEM_SHARED((NPAD, HID), _f32),
                          pltpu.VMEM_SHARED((NPAD, 16), _f32)]
                       + [pltpu.SemaphoreType.DMA] * 6),
    )(_edge_kernel)
    return kern(q_all, k_all, v_all, src_all, dst_all)


def _fuse_rel(w, b, rel, scale):
    bd = rel * scale[:, None, None]
    BD = (bd[:, :, None, :] * jnp.eye(H, dtype=_f32)[:, None, :, None]
          ).reshape(HID, HID)
    return w @ BD, b @ BD


def kernel(x_user, x_job, edge_uj, edge_ju, params):
    p = params
    sel = np.zeros((16, HID), np.float32)
    for h in range(H):
        sel[h, h * DH:(h + 1) * DH] = 1.0
    sel = jnp.asarray(sel)

    x_cat = jnp.concatenate([x_user, x_job], axis=0)
    in_w = jnp.stack([p["in_user_w"], p["in_job_w"]])
    in_b = jnp.stack([p["in_user_b"], p["in_job_b"]])
    x = _typed_matmul(x_cat, in_w, in_b, HID, "relu")

    src_all = jnp.concatenate([edge_uj[0], edge_ju[0]])
    dst_all = jnp.concatenate([edge_uj[1], edge_ju[1]])

    for l in range(L):
        s_uj = p[f"l{l}_prel_uj"] / np.sqrt(DH).astype(np.float32)
        s_ju = p[f"l{l}_prel_ju"] / np.sqrt(DH).astype(np.float32)
        ones = jnp.ones((H,), _f32)
        kw_u, kb_u = _fuse_rel(p[f"l{l}_K_user_w"], p[f"l{l}_K_user_b"],
                               p[f"l{l}_arel_uj"], s_uj)
        vw_u, vb_u = _fuse_rel(p[f"l{l}_V_user_w"], p[f"l{l}_V_user_b"],
                               p[f"l{l}_mrel_uj"], ones)
        kw_j, kb_j = _fuse_rel(p[f"l{l}_K_job_w"], p[f"l{l}_K_job_b"],
                               p[f"l{l}_arel_ju"], s_ju)
        vw_j, vb_j = _fuse_rel(p[f"l{l}_V_job_w"], p[f"l{l}_V_job_b"],
                               p[f"l{l}_mrel_ju"], ones)
        w2 = jnp.stack([
            jnp.concatenate([p[f"l{l}_Q_user_w"], kw_u, vw_u], axis=1),
            jnp.concatenate([p[f"l{l}_Q_job_w"], kw_j, vw_j], axis=1),
        ])
        b2 = jnp.stack([
            jnp.concatenate([p[f"l{l}_Q_user_b"], kb_u, vb_u]),
            jnp.concatenate([p[f"l{l}_Q_job_b"], kb_j, vb_j]),
        ])
        q_all, k_all, v_all = _qkv(x, w2, b2)
        msum, dsum = _edge_agg(q_all, k_all, v_all, src_all, dst_all)
        aw2 = jnp.stack([p[f"l{l}_A_user_w"], p[f"l{l}_A_job_w"]])
        ab2 = jnp.stack([p[f"l{l}_A_user_b"], p[f"l{l}_A_job_b"]])
        beta2 = jax.nn.sigmoid(jnp.stack([p[f"l{l}_skip_user"],
                                          p[f"l{l}_skip_job"]])).reshape(2, 1)
        x = _finish(msum, dsum, x, sel, aw2, ab2, beta2)

    y = _out_proj(x, p["out_w"], p["out_b"])
    return (y[:N], y[N:])


# speedup vs baseline: 19.5698x; 1.4215x over previous
import functools

import jax
import jax.numpy as jnp
import numpy as np
from jax import lax
from jax.experimental import pallas as pl
from jax.experimental.pallas import tpu as pltpu
from jax.experimental.pallas import tpu_sc as plsc

N = 10000
E = 320000
HID = 128
OUT = 64
H = 8
DH = 16
L = 2
NC, NS = 2, 16
EPT = E // NS
CH = 48
NCHUNK = (EPT // CH) & ~1
CHT = EPT - NCHUNK * CH
NPAD = 10240
NPT = NPAD // NS
RB = 1000
GRID = 2 * N // RB

_f32 = jnp.float32


def _mm_body(x_ref, w_ref, b_ref, o_ref, *, act):
    y = jnp.dot(x_ref[...], w_ref[0], preferred_element_type=_f32) + b_ref[0, 0]
    if act == "relu":
        y = jnp.maximum(y, 0.0)
    o_ref[...] = y


def _typed_matmul(x, w2, b2, nout, act):
    return pl.pallas_call(
        functools.partial(_mm_body, act=act),
        grid=(GRID,),
        in_specs=[
            pl.BlockSpec((RB, HID), lambda i: (i, 0)),
            pl.BlockSpec((1, HID, nout), lambda i: (i // (GRID // 2), 0, 0)),
            pl.BlockSpec((1, 1, nout), lambda i: (i // (GRID // 2), 0, 0)),
        ],
        out_specs=pl.BlockSpec((RB, nout), lambda i: (i, 0)),
        out_shape=jax.ShapeDtypeStruct((2 * N, nout), _f32),
    )(x, w2, b2.reshape(2, 1, nout))


def _qkv_body(x_ref, w_ref, b_ref, q_ref, k_ref, v_ref):
    y = jnp.dot(x_ref[...], w_ref[0], preferred_element_type=_f32) + b_ref[0, 0]
    q_ref[...] = y[:, :HID]
    k_ref[...] = y[:, HID:2 * HID]
    v_ref[...] = y[:, 2 * HID:]


def _qkv(x, w2, b2):
    outs = [jax.ShapeDtypeStruct((2 * N, HID), _f32)] * 3
    return pl.pallas_call(
        _qkv_body,
        grid=(GRID,),
        in_specs=[
            pl.BlockSpec((RB, HID), lambda i: (i, 0)),
            pl.BlockSpec((1, HID, 3 * HID), lambda i: (i // (GRID // 2), 0, 0)),
            pl.BlockSpec((1, 1, 3 * HID), lambda i: (i // (GRID // 2), 0, 0)),
        ],
        out_specs=[pl.BlockSpec((RB, HID), lambda i: (i, 0))] * 3,
        out_shape=outs,
    )(x, w2, b2.reshape(2, 1, 3 * HID))


def _finish_body(m_ref, d_ref, x_ref, sel_ref, aw_ref, ab_ref, bt_ref, o_ref):
    m = m_ref[0]
    den = d_ref[0]
    denb = jnp.dot(den, sel_ref[...], preferred_element_type=_f32) + 1e-16
    msg = m / denb
    hmid = jax.nn.gelu(msg)
    y = jnp.dot(hmid, aw_ref[0], preferred_element_type=_f32) + ab_ref[0, 0]
    beta = bt_ref[0, 0, 0]
    o_ref[...] = jnp.maximum(beta * y + (1.0 - beta) * x_ref[...], 0.0)


def _finish(msum, dsum, x, sel, aw2, ab2, beta2):
    half = GRID // 2
    return pl.pallas_call(
        _finish_body,
        grid=(GRID,),
        in_specs=[
            pl.BlockSpec((1, RB, HID), lambda i: (1 - i // half, i % half, 0)),
            pl.BlockSpec((1, RB, 16), lambda i: (1 - i // half, i % half, 0)),
            pl.BlockSpec((RB, HID), lambda i: (i, 0)),
            pl.BlockSpec((16, HID), lambda i: (0, 0)),
            pl.BlockSpec((1, HID, HID), lambda i: (i // half, 0, 0)),
            pl.BlockSpec((1, 1, HID), lambda i: (i // half, 0, 0)),
            pl.BlockSpec((1, 1, 1), lambda i: (i // half, 0, 0)),
        ],
        out_specs=pl.BlockSpec((RB, HID), lambda i: (i, 0)),
        out_shape=jax.ShapeDtypeStruct((2 * N, HID), _f32),
    )(msum, dsum, x, sel, aw2, ab2.reshape(2, 1, HID), beta2.reshape(2, 1, 1))


def _out_body(x_ref, w_ref, b_ref, o_ref):
    o_ref[...] = jnp.dot(x_ref[...], w_ref[...], preferred_element_type=_f32) + b_ref[...]


def _out_proj(x, w, b):
    return pl.pallas_call(
        _out_body,
        grid=(GRID,),
        in_specs=[
            pl.BlockSpec((RB, HID), lambda i: (i, 0)),
            pl.BlockSpec((HID, OUT), lambda i: (0, 0)),
            pl.BlockSpec((1, OUT), lambda i: (0, 0)),
        ],
        out_specs=pl.BlockSpec((RB, OUT), lambda i: (i, 0)),
        out_shape=jax.ShapeDtypeStruct((2 * N, OUT), _f32),
    )(x, w, b.reshape(1, OUT))


def _edge_kernel(q_hbm, k_hbm, v_hbm, src_hbm, dst_hbm, outm_hbm, outd_hbm,
                 srcb0, dstb0, dstqb0, qb0, kb0, vb0, exb0,
                 srcb1, dstb1, dstqb1, qb1, kb1, vb1, exb1,
                 srcbt, dstbt, dstqbt,
                 accm_sh, accd_sh,
                 gsem0, gsem1, gsem2, gsem3, gsem4, gsem5):
    c = lax.axis_index("c")
    s = lax.axis_index("s")
    zrow = jnp.zeros((16,), _f32)
    sets = ((srcb0, dstb0, dstqb0, qb0, kb0, vb0, exb0, (gsem0, gsem1, gsem2)),
            (srcb1, dstb1, dstqb1, qb1, kb1, vb1, exb1, (gsem3, gsem4, gsem5)))

    def _vb_zero(r, _):
        for j in range(HID // 16):
            vb0[r, pl.ds(j * 16, 16)] = zrow
        exb0[r, pl.ds(0, 16)] = zrow
        return 0
    lax.fori_loop(0, CH, _vb_zero, 0)
    row0 = s * NPT
    nzc = NPT // CH
    for t in range(nzc):
        pltpu.sync_copy(vb0, accm_sh.at[pl.ds(row0 + t * CH, CH)])
        pltpu.sync_copy(exb0, accd_sh.at[pl.ds(row0 + t * CH, CH)])
    rem = NPT - nzc * CH
    if rem:
        pltpu.sync_copy(vb0.at[pl.ds(0, rem)],
                        accm_sh.at[pl.ds(row0 + nzc * CH, rem)])
        pltpu.sync_copy(exb0.at[pl.ds(0, rem)],
                        accd_sh.at[pl.ds(row0 + nzc * CH, rem)])

    plsc.subcore_barrier()

    src_off = c * N
    dst_off = (1 - c) * N
    ebase0 = c * E + s * EPT
    lanes = lax.iota(jnp.int32, 16)

    def issue(ck, st):
        srcb, dstb, dstqb, qb, kb, vb, exb, sems = st
        ebase = ebase0 + ck * CH
        pltpu.sync_copy(src_hbm.at[pl.ds(ebase, CH)], srcb)
        pltpu.sync_copy(dst_hbm.at[pl.ds(ebase, CH)], dstb)
        for i in range(CH // 16):
            sl = pl.ds(i * 16, 16)
            srcb[sl] = srcb[sl] + src_off
            dstqb[sl] = dstb[sl] + dst_off
        pltpu.async_copy(q_hbm.at[dstqb], qb, sems[0])
        pltpu.async_copy(k_hbm.at[srcb], kb, sems[1])
        pltpu.async_copy(v_hbm.at[srcb], vb, sems[2])

    def drain(st):
        srcb, dstb, dstqb, qb, kb, vb, exb, sems = st
        pltpu.make_async_copy(q_hbm.at[dstqb], qb, sems[0]).wait()
        pltpu.make_async_copy(k_hbm.at[srcb], kb, sems[1]).wait()
        pltpu.make_async_copy(v_hbm.at[srcb], vb, sems[2]).wait()

    lane0 = lanes < 1

    def make_group_body(qb, kb, vb, exb):
        def edge_body(e, _):
            erow = jnp.full((16,), e, jnp.int32)
            for h in range(H):
                sl = pl.ds(h * DH, DH)
                s = jnp.sum(qb[e, sl] * kb[e, sl])
                ex = jnp.exp(jnp.full((16,), s, _f32))
                plsc.store_scatter(exb, [erow, jnp.full((16,), h, jnp.int32)],
                                   ex, mask=lane0)
                vb[e, sl] = vb[e, sl] * ex
            return 0
        return edge_body

    issue(0, sets[0])

    def pair_body(j, _):
        for b in (0, 1):
            ck = 2 * j + b
            st = sets[b]
            drain(st)

            @pl.when(ck < NCHUNK - 1)
            def _():
                issue(ck + 1, sets[1 - b])
            lax.fori_loop(0, CH, make_group_body(st[3], st[4], st[5], st[6]), 0)
            pltpu.sync_copy(st[5], accm_sh.at[st[1]], add=True)
            pltpu.sync_copy(st[6], accd_sh.at[st[1]], add=True)
        return 0
    lax.fori_loop(0, NCHUNK // 2, pair_body, 0)

    if CHT:
        ebase = ebase0 + NCHUNK * CH
        pltpu.sync_copy(src_hbm.at[pl.ds(ebase, CHT)], srcbt)
        pltpu.sync_copy(dst_hbm.at[pl.ds(ebase, CHT)], dstbt)
        for i in range(CHT // 16):
            sl = pl.ds(i * 16, 16)
            srcbt[sl] = srcbt[sl] + src_off
            dstqbt[sl] = dstbt[sl] + dst_off
        qbt, kbt, vbt = (qb0.at[pl.ds(0, CHT)], kb0.at[pl.ds(0, CHT)],
                         vb0.at[pl.ds(0, CHT)])
        exbt = exb0.at[pl.ds(0, CHT)]
        pltpu.async_copy(q_hbm.at[dstqbt], qbt, gsem0)
        pltpu.async_copy(k_hbm.at[srcbt], kbt, gsem1)
        pltpu.async_copy(v_hbm.at[srcbt], vbt, gsem2)
        pltpu.make_async_copy(q_hbm.at[dstqbt], qbt, gsem0).wait()
        pltpu.make_async_copy(k_hbm.at[srcbt], kbt, gsem1).wait()
        pltpu.make_async_copy(v_hbm.at[srcbt], vbt, gsem2).wait()
        lax.fori_loop(0, CHT, make_group_body(qb0, kb0, vb0, exb0), 0)
        pltpu.sync_copy(vb0.at[pl.ds(0, CHT)], accm_sh.at[dstbt], add=True)
        pltpu.sync_copy(exb0.at[pl.ds(0, CHT)], accd_sh.at[dstbt], add=True)

    plsc.subcore_barrier()
    pltpu.sync_copy(accm_sh.at[pl.ds(row0, NPT)],
                    outm_hbm.at[c, pl.ds(row0, NPT)])
    pltpu.sync_copy(accd_sh.at[pl.ds(row0, NPT)],
                    outd_hbm.at[c, pl.ds(row0, NPT)])


def _edge_agg(q_all, k_all, v_all, src_all, dst_all):
    mesh = plsc.VectorSubcoreMesh(core_axis_name="c", subcore_axis_name="s")
    idx_t = pltpu.VMEM((CH,), jnp.int32)
    buf_set = [idx_t, idx_t, idx_t,
               pltpu.VMEM((CH, HID), _f32), pltpu.VMEM((CH, HID), _f32),
               pltpu.VMEM((CH, HID), _f32), pltpu.VMEM((CH, 16), _f32)]
    kern = functools.partial(
        pl.kernel,
        mesh=mesh,
        compiler_params=pltpu.CompilerParams(
            needs_layout_passes=False, use_tc_tiling_on_sc=False),
        out_type=[jax.ShapeDtypeStruct((2, NPAD, HID), _f32),
                  jax.ShapeDtypeStruct((2, NPAD, 16), _f32)],
        scratch_types=(buf_set + buf_set
                       + [pltpu.VMEM((CHT,), jnp.int32)] * 3
                       + [pltpu.VMEM_SHARED((NPAD, HID), _f32),
                          pltpu.VMEM_SHARED((NPAD, 16), _f32)]
                       + [pltpu.SemaphoreType.DMA] * 6),
    )(_edge_kernel)
    return kern(q_all, k_all, v_all, src_all, dst_all)


def _fuse_rel(w, b, rel, scale):
    bd = rel * scale[:, None, None]
    BD = (bd[:, :, None, :] * jnp.eye(H, dtype=_f32)[:, None, :, None]
          ).reshape(HID, HID)
    return w @ BD, b @ BD


def kernel(x_user, x_job, edge_uj, edge_ju, params):
    p = params
    sel = np.zeros((16, HID), np.float32)
    for h in range(H):
        sel[h, h * DH:(h + 1) * DH] = 1.0
    sel = jnp.asarray(sel)

    x_cat = jnp.concatenate([x_user, x_job], axis=0)
    in_w = jnp.stack([p["in_user_w"], p["in_job_w"]])
    in_b = jnp.stack([p["in_user_b"], p["in_job_b"]])
    x = _typed_matmul(x_cat, in_w, in_b, HID, "relu")

    src_all = jnp.concatenate([edge_uj[0], edge_ju[0]])
    dst_all = jnp.concatenate([edge_uj[1], edge_ju[1]])

    for l in range(L):
        s_uj = p[f"l{l}_prel_uj"] / np.sqrt(DH).astype(np.float32)
        s_ju = p[f"l{l}_prel_ju"] / np.sqrt(DH).astype(np.float32)
        ones = jnp.ones((H,), _f32)
        kw_u, kb_u = _fuse_rel(p[f"l{l}_K_user_w"], p[f"l{l}_K_user_b"],
                               p[f"l{l}_arel_uj"], s_uj)
        vw_u, vb_u = _fuse_rel(p[f"l{l}_V_user_w"], p[f"l{l}_V_user_b"],
                               p[f"l{l}_mrel_uj"], ones)
        kw_j, kb_j = _fuse_rel(p[f"l{l}_K_job_w"], p[f"l{l}_K_job_b"],
                               p[f"l{l}_arel_ju"], s_ju)
        vw_j, vb_j = _fuse_rel(p[f"l{l}_V_job_w"], p[f"l{l}_V_job_b"],
                               p[f"l{l}_mrel_ju"], ones)
        w2 = jnp.stack([
            jnp.concatenate([p[f"l{l}_Q_user_w"], kw_u, vw_u], axis=1),
            jnp.concatenate([p[f"l{l}_Q_job_w"], kw_j, vw_j], axis=1),
        ])
        b2 = jnp.stack([
            jnp.concatenate([p[f"l{l}_Q_user_b"], kb_u, vb_u]),
            jnp.concatenate([p[f"l{l}_Q_job_b"], kb_j, vb_j]),
        ])
        q_all, k_all, v_all = _qkv(x, w2, b2)
        msum, dsum = _edge_agg(q_all, k_all, v_all, src_all, dst_all)
        aw2 = jnp.stack([p[f"l{l}_A_user_w"], p[f"l{l}_A_job_w"]])
        ab2 = jnp.stack([p[f"l{l}_A_user_b"], p[f"l{l}_A_job_b"]])
        beta2 = jax.nn.sigmoid(jnp.stack([p[f"l{l}_skip_user"],
                                          p[f"l{l}_skip_job"]])).reshape(2, 1)
        x = _finish(msum, dsum, x, sel, aw2, ab2, beta2)

    y = _out_proj(x, p["out_w"], p["out_b"])
    return (y[:N], y[N:])
